# baseline (speedup 1.0000x reference)
import functools

import jax
import jax.numpy as jnp
from jax import lax
from jax.experimental import pallas as pl
from jax.experimental.pallas import tpu as pltpu
from jax.experimental.pallas import tpu_sc as plsc

_BATCH = 262144
_DIM = 256
_LANES = 16
_NCHUNK = _DIM // _LANES
_ROWS_PER_BLK = 32


@functools.partial(jax.jit, static_argnums=(1, 2))
def _reverse_cols(inputs, num_cores, num_subcores):
    num_workers = num_cores * num_subcores
    rows_per_w = _BATCH // num_workers
    nblk = rows_per_w // _ROWS_PER_BLK
    nbuf = 4
    ngrp = nblk // nbuf
    mesh = plsc.VectorSubcoreMesh(
        core_axis_name="c", subcore_axis_name="s",
        num_cores=num_cores, num_subcores=num_subcores)

    buf_t = pltpu.VMEM((_ROWS_PER_BLK, _DIM), jnp.float32)

    @functools.partial(
        pl.kernel,
        out_type=jax.ShapeDtypeStruct((_BATCH, _DIM), jnp.float32),
        mesh=mesh,
        scratch_types=[
            [buf_t] * nbuf, [buf_t] * nbuf,
            [pltpu.SemaphoreType.DMA] * nbuf, [pltpu.SemaphoreType.DMA] * nbuf,
        ],
    )
    def body(in_hbm, out_hbm, inbufs, outbufs, in_sems, out_sems):
        wid = lax.axis_index("s") * num_cores + lax.axis_index("c")
        base = wid * rows_per_w

        def in_blk(i):
            return in_hbm.at[pl.ds(base + i * _ROWS_PER_BLK, _ROWS_PER_BLK)]

        def out_blk(i):
            return out_hbm.at[pl.ds(base + i * _ROWS_PER_BLK, _ROWS_PER_BLK)]

        for b in range(nbuf):
            pltpu.async_copy(in_blk(b), inbufs[b], in_sems[b])

        def grp(g, carry):
            for b in range(nbuf):
                i = g * nbuf + b
                inbuf, outbuf = inbufs[b], outbufs[b]
                pltpu.make_async_copy(in_blk(i), inbuf, in_sems[b]).wait()

                @pl.when(g > 0)
                def _():
                    pltpu.make_async_copy(outbuf, out_blk(i - nbuf),
                                          out_sems[b]).wait()

                @plsc.parallel_loop(0, _ROWS_PER_BLK, step=1)
                def row(r):
                    for c in range(_NCHUNK):
                        v = inbuf[r, pl.ds((_NCHUNK - 1 - c) * _LANES, _LANES)]
                        outbuf[r, pl.ds(c * _LANES, _LANES)] = lax.rev(v, (0,))

                pltpu.async_copy(outbuf, out_blk(i), out_sems[b])

                @pl.when(g < ngrp - 1)
                def _():
                    pltpu.async_copy(in_blk(i + nbuf), inbuf, in_sems[b])
            return carry

        lax.fori_loop(0, ngrp, grp, 0)

        for b in range(nbuf):
            pltpu.make_async_copy(outbufs[b], out_blk(nblk - nbuf + b),
                                  out_sems[b]).wait()

    return body(inputs)


def kernel(inputs, feat):
    info = plsc.get_sparse_core_info()
    out = _reverse_cols(inputs, info.num_cores, info.num_subcores)
    return (out, 0)
